# baseline (speedup 1.0000x reference)
import functools

import jax
import jax.numpy as jnp
from jax import lax
from jax.experimental import pallas as pl
from jax.experimental.pallas import tpu as pltpu
from jax.experimental.pallas import tpu_sc as plsc

VOCAB = 100000
D_MODEL = 64
MAX_SEQ = 200
BATCH = 4096

_DPAD = 128

_NC = 2
_NS = 16
_L = 16
_NW = _NC * _NS

_B_PER_W = BATCH // _NW
_CHUNK_B = 2
_ROWS = _CHUNK_B * MAX_SEQ
_N_CHUNK = _B_PER_W // _CHUNK_B
_GLEN = 80
_GATHERS = _ROWS // _GLEN


def _pe_table():
    even_i = jnp.arange(0, D_MODEL, 2).astype(jnp.float32)
    denominator = jnp.power(10000.0, even_i / D_MODEL)
    position = jnp.arange(MAX_SEQ).reshape(MAX_SEQ, 1).astype(jnp.float32)
    even_pe = jnp.sin(position / denominator)
    odd_pe = jnp.cos(position / denominator)
    return jnp.stack([even_pe, odd_pe], axis=2).reshape(MAX_SEQ, D_MODEL)


def kernel(x, embedding):
    idx = x.reshape(-1)
    emb = jnp.pad(embedding, ((0, 0), (0, _DPAD - D_MODEL)))
    pe = _pe_table()

    mesh = plsc.VectorSubcoreMesh(core_axis_name="c", subcore_axis_name="s")

    @functools.partial(
        pl.kernel,
        mesh=mesh,
        out_type=jax.ShapeDtypeStruct((BATCH * MAX_SEQ, D_MODEL), jnp.float32),
        scratch_types=[
            pltpu.VMEM((_ROWS,), jnp.int32),
            pltpu.VMEM((_ROWS, _DPAD), jnp.float32),
            pltpu.VMEM((_ROWS, D_MODEL), jnp.float32),
            pltpu.VMEM((MAX_SEQ, D_MODEL), jnp.float32),
            pltpu.SemaphoreType.DMA,
        ],
    )
    def sc_kernel(idx_hbm, emb_hbm, pe_hbm, out_hbm,
                  idx_v, buf_v, stg_v, pe_v, sem):
        wid = lax.axis_index("s") * _NC + lax.axis_index("c")
        pltpu.sync_copy(pe_hbm, pe_v)

        def chunk_body(c, carry):
            flat0 = (wid * _N_CHUNK + c) * _ROWS
            pltpu.sync_copy(idx_hbm.at[pl.ds(flat0, _ROWS)], idx_v)
            copies = [
                pltpu.async_copy(
                    emb_hbm.at[idx_v.at[pl.ds(j * _GLEN, _GLEN)]],
                    buf_v.at[pl.ds(j * _GLEN, _GLEN)],
                    sem,
                )
                for j in range(_GATHERS)
            ]
            for cpy in copies:
                cpy.wait()

            def add_body(r, c2):
                pr = lax.rem(r, MAX_SEQ)
                for j in range(D_MODEL // _L):
                    s = pl.ds(j * _L, _L)
                    stg_v[r, s] = buf_v[r, s] + pe_v[pr, s]
                return c2

            lax.fori_loop(0, _ROWS, add_body, 0)
            pltpu.sync_copy(stg_v, out_hbm.at[pl.ds(flat0, _ROWS)])
            return carry

        lax.fori_loop(0, _N_CHUNK, chunk_body, 0)

    out = sc_kernel(idx, emb, pe)
    return out.reshape(BATCH, MAX_SEQ, D_MODEL)


# speedup vs baseline: 1.0398x; 1.0398x over previous
import functools

import jax
import jax.numpy as jnp
from jax import lax
from jax.experimental import pallas as pl
from jax.experimental.pallas import tpu as pltpu
from jax.experimental.pallas import tpu_sc as plsc

VOCAB = 100000
D_MODEL = 64
MAX_SEQ = 200
BATCH = 4096

_DPAD = 128

_NC = 2
_NS = 16
_L = 16
_NW = _NC * _NS

_B_PER_W = BATCH // _NW
_CHUNK_B = 2
_ROWS = _CHUNK_B * MAX_SEQ
_N_CHUNK = _B_PER_W // _CHUNK_B
_GLEN = 80
_GATHERS = _ROWS // _GLEN


def _pe_table():
    even_i = jnp.arange(0, D_MODEL, 2).astype(jnp.float32)
    denominator = jnp.power(10000.0, even_i / D_MODEL)
    position = jnp.arange(MAX_SEQ).reshape(MAX_SEQ, 1).astype(jnp.float32)
    even_pe = jnp.sin(position / denominator)
    odd_pe = jnp.cos(position / denominator)
    return jnp.stack([even_pe, odd_pe], axis=2).reshape(MAX_SEQ, D_MODEL)


def kernel(x, embedding):
    idx = x.reshape(-1)
    emb = jnp.pad(embedding, ((0, 0), (0, _DPAD - D_MODEL)))
    pe = _pe_table()

    mesh = plsc.VectorSubcoreMesh(core_axis_name="c", subcore_axis_name="s")

    @functools.partial(
        pl.kernel,
        mesh=mesh,
        out_type=jax.ShapeDtypeStruct((BATCH * MAX_SEQ, _DPAD), jnp.float32),
        scratch_types=[
            pltpu.VMEM((_ROWS,), jnp.int32),
            pltpu.VMEM((_ROWS, _DPAD), jnp.float32),
            pltpu.VMEM((_ROWS, D_MODEL), jnp.float32),
            pltpu.VMEM((MAX_SEQ, D_MODEL), jnp.float32),
            pltpu.SemaphoreType.DMA,
        ],
    )
    def sc_kernel(idx_hbm, emb_hbm, pe_hbm, out_hbm,
                  idx_v, buf_v, stg_v, pe_v, sem):
        wid = lax.axis_index("s") * _NC + lax.axis_index("c")
        pltpu.sync_copy(pe_hbm, pe_v)

        def chunk_body(c, carry):
            flat0 = (wid * _N_CHUNK + c) * _ROWS
            pltpu.sync_copy(idx_hbm.at[pl.ds(flat0, _ROWS)], idx_v)
            copies = [
                pltpu.async_copy(
                    emb_hbm.at[idx_v.at[pl.ds(j * _GLEN, _GLEN)]],
                    buf_v.at[pl.ds(j * _GLEN, _GLEN)],
                    sem,
                )
                for j in range(_GATHERS)
            ]
            for cpy in copies:
                cpy.wait()

            def add_body(r, c2):
                pr = lax.rem(r, MAX_SEQ)
                for j in range(D_MODEL // _L):
                    s = pl.ds(j * _L, _L)
                    buf_v[r, s] = buf_v[r, s] + pe_v[pr, s]
                return c2

            lax.fori_loop(0, _ROWS, add_body, 0)
            pltpu.sync_copy(buf_v, out_hbm.at[pl.ds(flat0, _ROWS)])
            return carry

        lax.fori_loop(0, _N_CHUNK, chunk_body, 0)

    out = sc_kernel(idx, emb, pe)
    return out[:, :D_MODEL].reshape(BATCH, MAX_SEQ, D_MODEL)


# speedup vs baseline: 1.3963x; 1.3429x over previous
import functools

import jax
import jax.numpy as jnp
from jax import lax
from jax.experimental import pallas as pl
from jax.experimental.pallas import tpu as pltpu
from jax.experimental.pallas import tpu_sc as plsc

VOCAB = 100000
D_MODEL = 64
MAX_SEQ = 200
BATCH = 4096

_NC = 2
_NS = 16
_L = 16
_NW = _NC * _NS

_CPW = BATCH // _NW
_NBUF = 4
_QUADS = _CPW // _NBUF
_IDX_MINOR = 100
_GATHERS = MAX_SEQ // _IDX_MINOR


def _pe_table():
    even_i = jnp.arange(0, D_MODEL, 2).astype(jnp.float32)
    denominator = jnp.power(10000.0, even_i / D_MODEL)
    position = jnp.arange(MAX_SEQ).reshape(MAX_SEQ, 1).astype(jnp.float32)
    even_pe = jnp.sin(position / denominator)
    odd_pe = jnp.cos(position / denominator)
    return jnp.stack([even_pe, odd_pe], axis=2).reshape(MAX_SEQ, D_MODEL)


def kernel(x, embedding):
    idx = x.reshape(-1, _IDX_MINOR)
    pe = _pe_table()

    mesh = plsc.VectorSubcoreMesh(core_axis_name="c", subcore_axis_name="s")

    @functools.partial(
        pl.kernel,
        mesh=mesh,
        out_type=jax.ShapeDtypeStruct((BATCH * MAX_SEQ, D_MODEL), jnp.float32),
        compiler_params=pltpu.CompilerParams(use_tc_tiling_on_sc=False),
        scratch_types=[
            *[pltpu.VMEM((_GATHERS, _IDX_MINOR), jnp.int32)
              for _ in range(_NBUF)],
            *[pltpu.VMEM((MAX_SEQ, D_MODEL), jnp.float32)
              for _ in range(_NBUF)],
            pltpu.VMEM((MAX_SEQ, D_MODEL), jnp.float32),
            pltpu.SemaphoreType.DMA,
            pltpu.SemaphoreType.DMA,
        ],
    )
    def sc_kernel(idx_hbm, emb_hbm, pe_hbm, out_hbm,
                  i0, i1, i2, i3, b0, b1, b2, b3, pe_v, sem_g, sem_w):
        idxs = (i0, i1, i2, i3)
        bufs = (b0, b1, b2, b3)
        wid = lax.axis_index("s") * _NC + lax.axis_index("c")
        pltpu.sync_copy(pe_hbm, pe_v)

        def drain_write(k):
            pltpu.make_async_copy(
                bufs[k], out_hbm.at[pl.ds(k * MAX_SEQ, MAX_SEQ)], sem_w
            ).wait()

        def fire_gathers(k, chunk):
            return [
                pltpu.async_copy(
                    emb_hbm.at[idxs[k].at[j]],
                    bufs[k].at[pl.ds(j * _IDX_MINOR, _IDX_MINOR)],
                    sem_g,
                )
                for j in range(_GATHERS)
            ]

        def add_pe(k):
            def add_body(r, c2):
                for j in range(D_MODEL // _L):
                    s = pl.ds(j * _L, _L)
                    bufs[k][r, s] = bufs[k][r, s] + pe_v[r, s]
                return c2

            lax.fori_loop(0, MAX_SEQ, add_body, 0)

        def fire_write(k, chunk):
            pltpu.async_copy(
                bufs[k], out_hbm.at[pl.ds(chunk * MAX_SEQ, MAX_SEQ)], sem_w
            )

        for k in range(_NBUF):
            fire_write(k, wid * _CPW + k)

        def quad_body(q, carry):
            base = wid * _CPW + q * _NBUF
            for k in range(_NBUF):
                drain_write(k)
                pltpu.sync_copy(
                    idx_hbm.at[pl.ds((base + k) * _GATHERS, _GATHERS)],
                    idxs[k],
                )
            g = [None] * _NBUF
            g[0] = fire_gathers(0, base)
            g[1] = fire_gathers(1, base + 1)
            for k in range(_NBUF):
                for cpy in g[k]:
                    cpy.wait()
                if k + 2 < _NBUF:
                    g[k + 2] = fire_gathers(k + 2, base + k + 2)
                add_pe(k)
                fire_write(k, base + k)
            return carry

        lax.fori_loop(0, _QUADS, quad_body, 0)
        for k in range(_NBUF):
            drain_write(k)

    out = sc_kernel(idx, embedding, pe)
    return out.reshape(BATCH, MAX_SEQ, D_MODEL)


# speedup vs baseline: 1.3996x; 1.0024x over previous
import functools

import jax
import jax.numpy as jnp
from jax import lax
from jax.experimental import pallas as pl
from jax.experimental.pallas import tpu as pltpu
from jax.experimental.pallas import tpu_sc as plsc

VOCAB = 100000
D_MODEL = 64
MAX_SEQ = 200
BATCH = 4096

_NC = 2
_NS = 16
_L = 16
_NW = _NC * _NS

_CPW = BATCH // _NW
_NBUF = 4
_QUADS = _CPW // _NBUF
_SPLITS = (0, 104, MAX_SEQ)


def _pe_table():
    even_i = jnp.arange(0, D_MODEL, 2).astype(jnp.float32)
    denominator = jnp.power(10000.0, even_i / D_MODEL)
    position = jnp.arange(MAX_SEQ).reshape(MAX_SEQ, 1).astype(jnp.float32)
    even_pe = jnp.sin(position / denominator)
    odd_pe = jnp.cos(position / denominator)
    return jnp.stack([even_pe, odd_pe], axis=2).reshape(MAX_SEQ, D_MODEL)


def kernel(x, embedding):
    pe = _pe_table()

    mesh = plsc.VectorSubcoreMesh(core_axis_name="c", subcore_axis_name="s")

    @functools.partial(
        pl.kernel,
        mesh=mesh,
        out_type=jax.ShapeDtypeStruct((BATCH, MAX_SEQ, D_MODEL), jnp.float32),
        compiler_params=pltpu.CompilerParams(use_tc_tiling_on_sc=False),
        scratch_types=[
            *[pltpu.VMEM((MAX_SEQ,), jnp.int32) for _ in range(_NBUF)],
            *[pltpu.VMEM((MAX_SEQ, D_MODEL), jnp.float32)
              for _ in range(_NBUF)],
            pltpu.VMEM((MAX_SEQ, D_MODEL), jnp.float32),
            pltpu.SemaphoreType.DMA,
            pltpu.SemaphoreType.DMA,
        ],
    )
    def sc_kernel(x_hbm, emb_hbm, pe_hbm, out_hbm,
                  i0, i1, i2, i3, b0, b1, b2, b3, pe_v, sem_g, sem_w):
        idxs = (i0, i1, i2, i3)
        bufs = (b0, b1, b2, b3)
        wid = lax.axis_index("s") * _NC + lax.axis_index("c")
        pltpu.sync_copy(pe_hbm, pe_v)

        def drain_write(k):
            pltpu.make_async_copy(bufs[k], out_hbm.at[k], sem_w).wait()

        def fire_gathers(k):
            return [
                pltpu.async_copy(
                    emb_hbm.at[idxs[k].at[pl.ds(lo, hi - lo)]],
                    bufs[k].at[pl.ds(lo, hi - lo)],
                    sem_g,
                )
                for lo, hi in zip(_SPLITS[:-1], _SPLITS[1:])
            ]

        def add_pe(k):
            def add_body(r, c2):
                for j in range(D_MODEL // _L):
                    s = pl.ds(j * _L, _L)
                    bufs[k][r, s] = bufs[k][r, s] + pe_v[r, s]
                return c2

            lax.fori_loop(0, MAX_SEQ, add_body, 0)

        def fire_write(k, chunk):
            pltpu.async_copy(bufs[k], out_hbm.at[chunk], sem_w)

        for k in range(_NBUF):
            fire_write(k, wid * _CPW + k)

        def quad_body(q, carry):
            base = wid * _CPW + q * _NBUF
            for k in range(_NBUF):
                drain_write(k)
                pltpu.sync_copy(x_hbm.at[base + k], idxs[k])
            g = [None] * _NBUF
            g[0] = fire_gathers(0)
            g[1] = fire_gathers(1)
            for k in range(_NBUF):
                for cpy in g[k]:
                    cpy.wait()
                if k + 2 < _NBUF:
                    g[k + 2] = fire_gathers(k + 2)
                add_pe(k)
                fire_write(k, base + k)
            return carry

        lax.fori_loop(0, _QUADS, quad_body, 0)
        for k in range(_NBUF):
            drain_write(k)

    return sc_kernel(x, embedding, pe)


# speedup vs baseline: 2.2745x; 1.6251x over previous
import functools

import jax
import jax.numpy as jnp
from jax import lax
from jax.experimental import pallas as pl
from jax.experimental.pallas import tpu as pltpu
from jax.experimental.pallas import tpu_sc as plsc

VOCAB = 100000
D_MODEL = 64
MAX_SEQ = 200
BATCH = 4096

_NC = 2
_NS = 16
_L = 16
_NW = _NC * _NS

_CPW = BATCH // _NW
_NBUF = 4
_QUADS = _CPW // _NBUF
_SPLITS = (0, 104, MAX_SEQ)


def _pe_table():
    even_i = jnp.arange(0, D_MODEL, 2).astype(jnp.float32)
    denominator = jnp.power(10000.0, even_i / D_MODEL)
    position = jnp.arange(MAX_SEQ).reshape(MAX_SEQ, 1).astype(jnp.float32)
    even_pe = jnp.sin(position / denominator)
    odd_pe = jnp.cos(position / denominator)
    return jnp.stack([even_pe, odd_pe], axis=2).reshape(MAX_SEQ, D_MODEL)


def kernel(x, embedding):
    pe = _pe_table()

    mesh = plsc.VectorSubcoreMesh(core_axis_name="c", subcore_axis_name="s")

    @functools.partial(
        pl.kernel,
        mesh=mesh,
        out_type=jax.ShapeDtypeStruct((BATCH, MAX_SEQ, 128), jnp.float32),
        compiler_params=pltpu.CompilerParams(use_tc_tiling_on_sc=False),
        scratch_types=[
            *[pltpu.VMEM((MAX_SEQ,), jnp.int32) for _ in range(_NBUF)],
            *[pltpu.VMEM((MAX_SEQ, D_MODEL), jnp.float32)
              for _ in range(_NBUF)],
            pltpu.VMEM((MAX_SEQ, D_MODEL), jnp.float32),
            pltpu.SemaphoreType.DMA,
            pltpu.SemaphoreType.DMA,
        ],
    )
    def sc_kernel(x_hbm, emb_hbm, pe_hbm, out_hbm,
                  i0, i1, i2, i3, b0, b1, b2, b3, pe_v, sem_g, sem_w):
        idxs = (i0, i1, i2, i3)
        bufs = (b0, b1, b2, b3)
        wid = lax.axis_index("s") * _NC + lax.axis_index("c")
        pltpu.sync_copy(pe_hbm, pe_v)

        def drain_write(k):
            pltpu.make_async_copy(
                bufs[k], out_hbm.at[k, :, pl.ds(0, D_MODEL)], sem_w
            ).wait()

        def fire_gathers(k):
            return [
                pltpu.async_copy(
                    emb_hbm.at[idxs[k].at[pl.ds(lo, hi - lo)]],
                    bufs[k].at[pl.ds(lo, hi - lo)],
                    sem_g,
                )
                for lo, hi in zip(_SPLITS[:-1], _SPLITS[1:])
            ]

        def add_pe(k):
            def add_body(r, c2):
                for j in range(D_MODEL // _L):
                    s = pl.ds(j * _L, _L)
                    bufs[k][r, s] = bufs[k][r, s] + pe_v[r, s]
                return c2

            lax.fori_loop(0, MAX_SEQ, add_body, 0)

        def fire_write(k, chunk):
            pltpu.async_copy(
                bufs[k], out_hbm.at[chunk, :, pl.ds(0, D_MODEL)], sem_w
            )

        for k in range(_NBUF):
            fire_write(k, wid * _CPW + k)

        def quad_body(q, carry):
            base = wid * _CPW + q * _NBUF
            for k in range(_NBUF):
                drain_write(k)
                pltpu.sync_copy(x_hbm.at[base + k], idxs[k])
            g = [None] * _NBUF
            g[0] = fire_gathers(0)
            g[1] = fire_gathers(1)
            for k in range(_NBUF):
                for cpy in g[k]:
                    cpy.wait()
                if k + 2 < _NBUF:
                    g[k + 2] = fire_gathers(k + 2)
                add_pe(k)
                fire_write(k, base + k)
            return carry

        lax.fori_loop(0, _QUADS, quad_body, 0)
        for k in range(_NBUF):
            drain_write(k)

    out = sc_kernel(x, embedding, pe)
    return out[:, :, :D_MODEL]


# speedup vs baseline: 2.4173x; 1.0628x over previous
import functools

import jax
import jax.numpy as jnp
from jax import lax
from jax.experimental import pallas as pl
from jax.experimental.pallas import tpu as pltpu
from jax.experimental.pallas import tpu_sc as plsc

VOCAB = 100000
D_MODEL = 64
MAX_SEQ = 200
BATCH = 4096

_NC = 2
_NS = 16
_L = 16
_NW = _NC * _NS

_CPW = BATCH // _NW
_NBUF = 4
_QUADS = _CPW // _NBUF
_SPLITS = (0, 104, MAX_SEQ)


def _pe_table():
    even_i = jnp.arange(0, D_MODEL, 2).astype(jnp.float32)
    denominator = jnp.power(10000.0, even_i / D_MODEL)
    position = jnp.arange(MAX_SEQ).reshape(MAX_SEQ, 1).astype(jnp.float32)
    even_pe = jnp.sin(position / denominator)
    odd_pe = jnp.cos(position / denominator)
    return jnp.stack([even_pe, odd_pe], axis=2).reshape(MAX_SEQ, D_MODEL)


def kernel(x, embedding):
    pe = _pe_table()

    mesh = plsc.VectorSubcoreMesh(core_axis_name="c", subcore_axis_name="s")

    @functools.partial(
        pl.kernel,
        mesh=mesh,
        out_type=jax.ShapeDtypeStruct((BATCH, MAX_SEQ, 128), jnp.float32),
        compiler_params=pltpu.CompilerParams(use_tc_tiling_on_sc=False),
        scratch_types=[
            pltpu.VMEM((_NBUF, MAX_SEQ), jnp.int32),
            *[pltpu.VMEM((MAX_SEQ, D_MODEL), jnp.float32)
              for _ in range(_NBUF)],
            pltpu.VMEM((MAX_SEQ, D_MODEL), jnp.float32),
            pltpu.SemaphoreType.DMA,
            pltpu.SemaphoreType.DMA,
        ],
    )
    def sc_kernel(x_hbm, emb_hbm, pe_hbm, out_hbm,
                  idx_v, b0, b1, b2, b3, pe_v, sem_g, sem_w):
        bufs = (b0, b1, b2, b3)
        wid = lax.axis_index("s") * _NC + lax.axis_index("c")
        pltpu.sync_copy(pe_hbm, pe_v)

        def drain_write(k):
            pltpu.make_async_copy(
                bufs[k], out_hbm.at[k, :, pl.ds(0, D_MODEL)], sem_w
            ).wait()

        def fire_gathers(k):
            return [
                pltpu.async_copy(
                    emb_hbm.at[idx_v.at[k, pl.ds(lo, hi - lo)]],
                    bufs[k].at[pl.ds(lo, hi - lo)],
                    sem_g,
                )
                for lo, hi in zip(_SPLITS[:-1], _SPLITS[1:])
            ]

        def add_pe(k):
            def add_body(r, c2):
                for j in range(D_MODEL // _L):
                    s = pl.ds(j * _L, _L)
                    bufs[k][r, s] = bufs[k][r, s] + pe_v[r, s]
                return c2

            lax.fori_loop(0, MAX_SEQ, add_body, 0)

        def fire_write(k, chunk):
            pltpu.async_copy(
                bufs[k], out_hbm.at[chunk, :, pl.ds(0, D_MODEL)], sem_w
            )

        for k in range(_NBUF):
            fire_write(k, wid * _CPW + k)

        def quad_body(q, carry):
            base = wid * _CPW + q * _NBUF
            for k in range(_NBUF):
                drain_write(k)
            pltpu.sync_copy(x_hbm.at[pl.ds(base, _NBUF)], idx_v)
            g = [fire_gathers(k) for k in range(_NBUF)]
            for k in range(_NBUF):
                for cpy in g[k]:
                    cpy.wait()
                add_pe(k)
                fire_write(k, base + k)
            return carry

        lax.fori_loop(0, _QUADS, quad_body, 0)
        for k in range(_NBUF):
            drain_write(k)

    out = sc_kernel(x, embedding, pe)
    return out[:, :, :D_MODEL]
